# Initial kernel scaffold; baseline (speedup 1.0000x reference)
import jax
import jax.numpy as jnp
from jax.experimental import pallas as pl


def kernel(x, edge_index, W1, b1, W2, b2, Wg, bg):
    raise NotImplementedError("write your pallas kernel here")


# speedup vs baseline: 6.6493x; 6.6493x over previous
import functools

import jax
import jax.numpy as jnp
from jax import lax
from jax.experimental import pallas as pl
from jax.experimental.pallas import tpu as pltpu
from jax.experimental.pallas import tpu_sc as plsc

SCALE = 1.8

NC = 2
NS = 16
NW = NC * NS
LANES = 16

CH = 128
HW = 128


def _sc_mesh():
    return plsc.VectorSubcoreMesh(core_axis_name="c", subcore_axis_name="s")


def _fill_const(ref, rows, vec16):
    def row(i, carry):
        def grp(k, carry2):
            ref[i, pl.ds(k * LANES, LANES)] = vec16
            return carry2
        return lax.fori_loop(0, HW // LANES, grp, carry)
    lax.fori_loop(0, rows, row, 0)


def _deg_kernel(dst_ch, n_pad):
    nchunk = dst_ch.shape[0]
    cpc = nchunk // NC
    cpw = cpc // NS
    rpw = n_pad // NS
    zrows = 128

    def body(dst_hbm, degp_hbm, idx2d, ones_v, zb, hist_sp):
        c = lax.axis_index("c")
        s = lax.axis_index("s")
        pltpu.sync_copy(dst_hbm.at[pl.ds(c * cpc + s * cpw, cpw)], idx2d)

        _fill_const(ones_v, CH, jnp.ones((LANES,), jnp.float32))
        _fill_const(zb, zrows, jnp.zeros((LANES,), jnp.float32))

        def zinit(t, carry):
            pltpu.sync_copy(zb, hist_sp.at[pl.ds(s * rpw + t * zrows, zrows)])
            return carry

        lax.fori_loop(0, rpw // zrows, zinit, 0)
        plsc.subcore_barrier()

        def astep(j, carry):
            pltpu.sync_copy(ones_v, hist_sp.at[idx2d.at[j]], add=True)
            return carry

        lax.fori_loop(0, cpw, astep, 0)
        plsc.subcore_barrier()
        pltpu.sync_copy(hist_sp.at[pl.ds(s * rpw, rpw)],
                        degp_hbm.at[c].at[pl.ds(s * rpw, rpw)])

    return pl.kernel(
        body,
        out_type=jax.ShapeDtypeStruct((NC, n_pad, HW), jnp.float32),
        mesh=_sc_mesh(),
        scratch_types=[
            pltpu.VMEM((cpw, CH), jnp.int32),
            pltpu.VMEM((CH, HW), jnp.float32),
            pltpu.VMEM((zrows, HW), jnp.float32),
            pltpu.VMEM_SHARED((n_pad, HW), jnp.float32),
        ],
    )(dst_ch)


def _dense_kernel(x, degp, Wc1, c1, W2, b2, Wg, n, n_pad, blk):
    d = x.shape[1]
    nblk = n_pad // blk

    def body(x_ref, degp_ref, wc1_ref, c1_ref, w2_ref, b2_ref, wg_ref,
             o1a, o1b, o2a, o2b):
        i = pl.program_id(0)
        deg = degp_ref[0, :, 0] + degp_ref[1, :, 0] + 1.0
        row = i * blk + lax.broadcasted_iota(jnp.int32, (blk, 1), 0)
        dis = jnp.where(row < n, lax.rsqrt(deg)[:, None], 0.0)
        xb = x_ref[...]
        t1 = (jnp.dot(xb, wc1_ref[...], preferred_element_type=jnp.float32)
              + c1_ref[...]) * dis
        h2 = jnp.dot(xb, w2_ref[...], preferred_element_type=jnp.float32) \
            + b2_ref[...]
        ss = jnp.sum(h2 * h2, axis=1, keepdims=True)
        h2n = h2 * (SCALE / jnp.maximum(jnp.sqrt(ss), 1e-12))
        t2 = jnp.dot(h2n, wg_ref[...], preferred_element_type=jnp.float32) \
            * dis
        h = d // 2
        o1a[...] = t1[:, :h]
        o1b[...] = t1[:, h:]
        o2a[...] = t2[:, :h]
        o2b[...] = t2[:, h:]

    out_sl = jax.ShapeDtypeStruct((n_pad, d // 2), jnp.float32)
    return pl.pallas_call(
        body,
        grid=(nblk,),
        in_specs=[
            pl.BlockSpec((blk, d), lambda i: (i, 0)),
            pl.BlockSpec((NC, blk, HW), lambda i: (0, i, 0)),
            pl.BlockSpec((d, d), lambda i: (0, 0)),
            pl.BlockSpec((1, d), lambda i: (0, 0)),
            pl.BlockSpec((d, d), lambda i: (0, 0)),
            pl.BlockSpec((1, d), lambda i: (0, 0)),
            pl.BlockSpec((d, d), lambda i: (0, 0)),
        ],
        out_specs=[pl.BlockSpec((blk, d // 2), lambda i: (i, 0))] * 4,
        out_shape=[out_sl] * 4,
    )(x, degp, Wc1, c1, W2, b2, Wg)


def _fold_kernel(W1, Wg, b1):
    d = W1.shape[0]

    def body(w1_ref, wg_ref, b1_ref, wc_ref, c1_ref):
        wg = wg_ref[...]
        wc_ref[...] = jnp.dot(w1_ref[...], wg, preferred_element_type=jnp.float32)
        c1_ref[...] = jnp.dot(b1_ref[...], wg, preferred_element_type=jnp.float32)

    return pl.pallas_call(
        body,
        out_shape=[
            jax.ShapeDtypeStruct((d, d), jnp.float32),
            jax.ShapeDtypeStruct((1, d), jnp.float32),
        ],
    )(W1, Wg, b1.reshape(1, d))


def _msg_kernel(src_ch, dst_ch, t0, t1, t2, t3, n_pad):
    nchunk = src_ch.shape[0]
    cpw = nchunk // NS
    rpw = n_pad // NS
    hw = t0.shape[1]

    def body(src_hbm, dst_hbm, t0_h, t1_h, t2_h, t3_h,
             a0_h, a1_h, a2_h, a3_h,
             sidx2d, didx2d, rows_v, sem, acc_sp):
        c = lax.axis_index("c")
        s = lax.axis_index("s")
        pltpu.sync_copy(src_hbm.at[pl.ds(s * cpw, cpw)], sidx2d)
        pltpu.sync_copy(dst_hbm.at[pl.ds(s * cpw, cpw)], didx2d)

        t_refs = (t0_h, t1_h, t2_h, t3_h)
        a_refs = (a0_h, a1_h, a2_h, a3_h)

        def run_slice(t_ref, a_ref):
            pltpu.sync_copy(t_ref.at[pl.ds(s * rpw, rpw)],
                            acc_sp.at[pl.ds(s * rpw, rpw)])
            plsc.subcore_barrier()

            def per_chunk(j, carry):
                pltpu.async_copy(t_ref.at[sidx2d.at[j]], rows_v, sem).wait()
                pltpu.sync_copy(rows_v, acc_sp.at[didx2d.at[j]], add=True)
                return carry

            lax.fori_loop(0, cpw, per_chunk, 0)
            plsc.subcore_barrier()
            pltpu.sync_copy(acc_sp.at[pl.ds(s * rpw, rpw)],
                            a_ref.at[pl.ds(s * rpw, rpw)])
            plsc.subcore_barrier()

        for sl in range(2):
            for cc in range(NC):
                @pl.when(c == cc)
                def _():
                    run_slice(t_refs[2 * cc + sl], a_refs[2 * cc + sl])

    out_sl = jax.ShapeDtypeStruct((n_pad, hw), jnp.float32)
    return pl.kernel(
        body,
        out_type=[out_sl] * 4,
        mesh=_sc_mesh(),
        scratch_types=[
            pltpu.VMEM((cpw, CH), jnp.int32),
            pltpu.VMEM((cpw, CH), jnp.int32),
            pltpu.VMEM((CH, hw), jnp.float32),
            pltpu.SemaphoreType.DMA,
            pltpu.VMEM_SHARED((n_pad, hw), jnp.float32),
        ],
    )(src_ch, dst_ch, t0, t1, t2, t3)


def _epilogue_kernel(a0, a1, a2, a3, degp, bg, n, n_pad, blk):
    hw = a0.shape[1]
    d = 2 * hw
    nblk = n_pad // blk

    def body(a0_r, a1_r, a2_r, a3_r, degp_ref, bg_ref, x_out, h_out):
        deg = degp_ref[0, :, 0] + degp_ref[1, :, 0] + 1.0
        dis = lax.rsqrt(deg)[:, None]
        bg = bg_ref[...]
        x_out[...] = dis * jnp.concatenate([a0_r[...], a1_r[...]], axis=1) + bg
        h_out[...] = dis * jnp.concatenate([a2_r[...], a3_r[...]], axis=1) + bg

    out = jax.ShapeDtypeStruct((n, d), jnp.float32)
    return pl.pallas_call(
        body,
        grid=(nblk,),
        in_specs=[pl.BlockSpec((blk, hw), lambda i: (i, 0))] * 4 + [
            pl.BlockSpec((NC, blk, HW), lambda i: (0, i, 0)),
            pl.BlockSpec((1, d), lambda i: (0, 0)),
        ],
        out_specs=[pl.BlockSpec((blk, d), lambda i: (i, 0))] * 2,
        out_shape=[out, out],
    )(a0, a1, a2, a3, degp, bg.reshape(1, d))


def kernel(x, edge_index, W1, b1, W2, b2, Wg, bg):
    n, d = x.shape
    e = edge_index.shape[1]
    blk = 256
    n_pad = ((n + blk - 1) // blk) * blk
    epw_align = NW * CH
    e_pad = ((e + epw_align - 1) // epw_align) * epw_align

    src = edge_index[0]
    dst = edge_index[1]
    padv = jnp.full((e_pad - e,), n, dtype=jnp.int32)
    src_p = jnp.concatenate([src, padv])
    dst_p = jnp.concatenate([dst, padv])

    dst_ch = dst_p.reshape(e_pad // CH, CH)
    degp = _deg_kernel(dst_ch, n_pad)

    Wc1, c1 = _fold_kernel(W1, Wg, b1)
    t0, t1, t2, t3 = _dense_kernel(x, degp, Wc1, c1, W2, b2.reshape(1, d),
                                   Wg, n, n_pad, blk)

    src_ch = src_p.reshape(e_pad // CH, CH)
    dst_ch = dst_p.reshape(e_pad // CH, CH)
    a0, a1, a2, a3 = _msg_kernel(src_ch, dst_ch, t0, t1, t2, t3, n_pad)

    x_, h = _epilogue_kernel(a0, a1, a2, a3, degp, bg, n, n_pad, blk)
    return (h, x_)


# speedup vs baseline: 7.3120x; 1.0997x over previous
import functools

import jax
import jax.numpy as jnp
from jax import lax
from jax.experimental import pallas as pl
from jax.experimental.pallas import tpu as pltpu
from jax.experimental.pallas import tpu_sc as plsc

SCALE = 1.8

NC = 2
NS = 16
NW = NC * NS
LANES = 16

CH = 128
HW = 128


def _sc_mesh():
    return plsc.VectorSubcoreMesh(core_axis_name="c", subcore_axis_name="s")


def _fill_const(ref, rows, vec16):
    def row(i, carry):
        def grp(k, carry2):
            ref[i, pl.ds(k * LANES, LANES)] = vec16
            return carry2
        return lax.fori_loop(0, HW // LANES, grp, carry)
    lax.fori_loop(0, rows, row, 0)


def _deg_kernel(dst_ch, n_pad):
    nchunk = dst_ch.shape[0]
    cpc = nchunk // NC
    cpw = cpc // NS
    rpw = n_pad // NS
    zrows = 128
    KB = 8

    def body(dst_hbm, degp_hbm, idx2d, ones_v, zb, dsem, hist_sp):
        c = lax.axis_index("c")
        s = lax.axis_index("s")
        pltpu.sync_copy(dst_hbm.at[pl.ds(c * cpc + s * cpw, cpw)], idx2d)

        _fill_const(ones_v, CH, jnp.ones((LANES,), jnp.float32))
        _fill_const(zb, zrows, jnp.zeros((LANES,), jnp.float32))

        def zinit(t, carry):
            pltpu.sync_copy(zb, hist_sp.at[pl.ds(s * rpw + t * zrows, zrows)])
            return carry

        lax.fori_loop(0, rpw // zrows, zinit, 0)
        plsc.subcore_barrier()

        def astep(g, carry):
            base = g * KB
            for b in range(KB):
                pltpu.async_copy(ones_v, hist_sp.at[idx2d.at[base + b]],
                                 dsem, add=True)
            for b in range(KB):
                pltpu.make_async_copy(ones_v, hist_sp.at[idx2d.at[base + b]],
                                      dsem).wait()
            return carry

        lax.fori_loop(0, cpw // KB, astep, 0)
        plsc.subcore_barrier()
        pltpu.sync_copy(hist_sp.at[pl.ds(s * rpw, rpw)],
                        degp_hbm.at[c].at[pl.ds(s * rpw, rpw)])

    return pl.kernel(
        body,
        out_type=jax.ShapeDtypeStruct((NC, n_pad, HW), jnp.float32),
        mesh=_sc_mesh(),
        scratch_types=[
            pltpu.VMEM((cpw, CH), jnp.int32),
            pltpu.VMEM((CH, HW), jnp.float32),
            pltpu.VMEM((zrows, HW), jnp.float32),
            pltpu.SemaphoreType.DMA,
            pltpu.VMEM_SHARED((n_pad, HW), jnp.float32),
        ],
    )(dst_ch)


def _dense_kernel(x, degp, Wc1, c1, W2, b2, Wg, n, n_pad, blk):
    d = x.shape[1]
    nblk = n_pad // blk

    def body(x_ref, degp_ref, wc1_ref, c1_ref, w2_ref, b2_ref, wg_ref,
             o1a, o1b, o2a, o2b):
        i = pl.program_id(0)
        deg = degp_ref[0, :, 0] + degp_ref[1, :, 0] + 1.0
        row = i * blk + lax.broadcasted_iota(jnp.int32, (blk, 1), 0)
        dis = jnp.where(row < n, lax.rsqrt(deg)[:, None], 0.0)
        xb = x_ref[...]
        t1 = (jnp.dot(xb, wc1_ref[...], preferred_element_type=jnp.float32)
              + c1_ref[...]) * dis
        h2 = jnp.dot(xb, w2_ref[...], preferred_element_type=jnp.float32) \
            + b2_ref[...]
        ss = jnp.sum(h2 * h2, axis=1, keepdims=True)
        h2n = h2 * (SCALE / jnp.maximum(jnp.sqrt(ss), 1e-12))
        t2 = jnp.dot(h2n, wg_ref[...], preferred_element_type=jnp.float32) \
            * dis
        h = d // 2
        o1a[...] = t1[:, :h]
        o1b[...] = t1[:, h:]
        o2a[...] = t2[:, :h]
        o2b[...] = t2[:, h:]

    out_sl = jax.ShapeDtypeStruct((n_pad, d // 2), jnp.float32)
    return pl.pallas_call(
        body,
        grid=(nblk,),
        in_specs=[
            pl.BlockSpec((blk, d), lambda i: (i, 0)),
            pl.BlockSpec((NC, blk, HW), lambda i: (0, i, 0)),
            pl.BlockSpec((d, d), lambda i: (0, 0)),
            pl.BlockSpec((1, d), lambda i: (0, 0)),
            pl.BlockSpec((d, d), lambda i: (0, 0)),
            pl.BlockSpec((1, d), lambda i: (0, 0)),
            pl.BlockSpec((d, d), lambda i: (0, 0)),
        ],
        out_specs=[pl.BlockSpec((blk, d // 2), lambda i: (i, 0))] * 4,
        out_shape=[out_sl] * 4,
    )(x, degp, Wc1, c1, W2, b2, Wg)


def _fold_kernel(W1, Wg, b1):
    d = W1.shape[0]

    def body(w1_ref, wg_ref, b1_ref, wc_ref, c1_ref):
        wg = wg_ref[...]
        wc_ref[...] = jnp.dot(w1_ref[...], wg, preferred_element_type=jnp.float32)
        c1_ref[...] = jnp.dot(b1_ref[...], wg, preferred_element_type=jnp.float32)

    return pl.pallas_call(
        body,
        out_shape=[
            jax.ShapeDtypeStruct((d, d), jnp.float32),
            jax.ShapeDtypeStruct((1, d), jnp.float32),
        ],
    )(W1, Wg, b1.reshape(1, d))


def _msg_kernel(src_ch, dst_ch, t0, t1, t2, t3, n_pad):
    nchunk = src_ch.shape[0]
    cpw = nchunk // NS
    rpw = n_pad // NS
    hw = t0.shape[1]
    NB = 2
    PH = 2
    pc = cpw // PH
    G = pc // NB

    def body(src_hbm, dst_hbm, t0_h, t1_h, t2_h, t3_h,
             a0_h, a1_h, a2_h, a3_h,
             sidx, didx, r0, r1, g0, g1, s0, s1, acc_sp):
        c = lax.axis_index("c")
        s = lax.axis_index("s")
        rows = (r0, r1)
        gsem = (g0, g1)
        ssem = (s0, s1)

        t_refs = (t0_h, t1_h, t2_h, t3_h)
        a_refs = (a0_h, a1_h, a2_h, a3_h)

        def run_slice(t_ref, a_ref):
            pltpu.sync_copy(t_ref.at[pl.ds(s * rpw, rpw)],
                            acc_sp.at[pl.ds(s * rpw, rpw)])
            plsc.subcore_barrier()

            for p in range(PH):
                base_hbm = s * cpw + p * pc
                pltpu.sync_copy(src_hbm.at[pl.ds(base_hbm, pc)], sidx)
                pltpu.sync_copy(dst_hbm.at[pl.ds(base_hbm, pc)], didx)

                for b in range(NB):
                    pltpu.async_copy(t_ref.at[sidx.at[b]], rows[b], gsem[b])

                def grp(g, carry):
                    base = g * NB
                    for b in range(NB):
                        j = base + b
                        pltpu.make_async_copy(t_ref.at[sidx.at[j]],
                                              rows[b], gsem[b]).wait()
                        pltpu.async_copy(rows[b], acc_sp.at[didx.at[j]],
                                         ssem[b], add=True)
                    for b in range(NB):
                        j = base + b
                        pltpu.make_async_copy(rows[b],
                                              acc_sp.at[didx.at[j]],
                                              ssem[b]).wait()

                        @pl.when(g < G - 1)
                        def _():
                            pltpu.async_copy(t_ref.at[sidx.at[j + NB]],
                                             rows[b], gsem[b])
                    return carry

                lax.fori_loop(0, G, grp, 0)

            plsc.subcore_barrier()
            pltpu.sync_copy(acc_sp.at[pl.ds(s * rpw, rpw)],
                            a_ref.at[pl.ds(s * rpw, rpw)])
            plsc.subcore_barrier()

        for sl in range(2):
            for cc in range(NC):
                @pl.when(c == cc)
                def _():
                    run_slice(t_refs[2 * cc + sl], a_refs[2 * cc + sl])

    out_sl = jax.ShapeDtypeStruct((n_pad, hw), jnp.float32)
    return pl.kernel(
        body,
        out_type=[out_sl] * 4,
        mesh=_sc_mesh(),
        scratch_types=[
            pltpu.VMEM((pc, CH), jnp.int32),
            pltpu.VMEM((pc, CH), jnp.int32),
            pltpu.VMEM((CH, hw), jnp.float32),
            pltpu.VMEM((CH, hw), jnp.float32),
            pltpu.SemaphoreType.DMA,
            pltpu.SemaphoreType.DMA,
            pltpu.SemaphoreType.DMA,
            pltpu.SemaphoreType.DMA,
            pltpu.VMEM_SHARED((n_pad, hw), jnp.float32),
        ],
    )(src_ch, dst_ch, t0, t1, t2, t3)


def _epilogue_kernel(a0, a1, a2, a3, degp, bg, n, n_pad, blk):
    hw = a0.shape[1]
    d = 2 * hw
    nblk = n_pad // blk

    def body(a0_r, a1_r, a2_r, a3_r, degp_ref, bg_ref, x_out, h_out):
        deg = degp_ref[0, :, 0] + degp_ref[1, :, 0] + 1.0
        dis = lax.rsqrt(deg)[:, None]
        bg = bg_ref[...]
        x_out[...] = dis * jnp.concatenate([a0_r[...], a1_r[...]], axis=1) + bg
        h_out[...] = dis * jnp.concatenate([a2_r[...], a3_r[...]], axis=1) + bg

    out = jax.ShapeDtypeStruct((n, d), jnp.float32)
    return pl.pallas_call(
        body,
        grid=(nblk,),
        in_specs=[pl.BlockSpec((blk, hw), lambda i: (i, 0))] * 4 + [
            pl.BlockSpec((NC, blk, HW), lambda i: (0, i, 0)),
            pl.BlockSpec((1, d), lambda i: (0, 0)),
        ],
        out_specs=[pl.BlockSpec((blk, d), lambda i: (i, 0))] * 2,
        out_shape=[out, out],
    )(a0, a1, a2, a3, degp, bg.reshape(1, d))


def kernel(x, edge_index, W1, b1, W2, b2, Wg, bg):
    n, d = x.shape
    e = edge_index.shape[1]
    blk = 256
    n_pad = ((n + blk - 1) // blk) * blk
    epw_align = NW * CH
    e_pad = ((e + epw_align - 1) // epw_align) * epw_align

    src = edge_index[0]
    dst = edge_index[1]
    padv = jnp.full((e_pad - e,), n, dtype=jnp.int32)
    src_p = jnp.concatenate([src, padv])
    dst_p = jnp.concatenate([dst, padv])

    dst_ch = dst_p.reshape(e_pad // CH, CH)
    degp = _deg_kernel(dst_ch, n_pad)

    Wc1, c1 = _fold_kernel(W1, Wg, b1)
    t0, t1, t2, t3 = _dense_kernel(x, degp, Wc1, c1, W2, b2.reshape(1, d),
                                   Wg, n, n_pad, blk)

    src_ch = src_p.reshape(e_pad // CH, CH)
    dst_ch = dst_p.reshape(e_pad // CH, CH)
    a0, a1, a2, a3 = _msg_kernel(src_ch, dst_ch, t0, t1, t2, t3, n_pad)

    x_, h = _epilogue_kernel(a0, a1, a2, a3, degp, bg, n, n_pad, blk)
    return (h, x_)


# speedup vs baseline: 7.7149x; 1.0551x over previous
import functools

import jax
import jax.numpy as jnp
from jax import lax
from jax.experimental import pallas as pl
from jax.experimental.pallas import tpu as pltpu
from jax.experimental.pallas import tpu_sc as plsc

SCALE = 1.8

NC = 2
NS = 16
NW = NC * NS
LANES = 16

CH = 128
HW = 128


def _sc_mesh():
    return plsc.VectorSubcoreMesh(core_axis_name="c", subcore_axis_name="s")


def _fill_const(ref, rows, vec16):
    def row(i, carry):
        def grp(k, carry2):
            ref[i, pl.ds(k * LANES, LANES)] = vec16
            return carry2
        return lax.fori_loop(0, HW // LANES, grp, carry)
    lax.fori_loop(0, rows, row, 0)


def _deg_kernel(dst_ch, n_pad):
    nchunk = dst_ch.shape[0]
    cpc = nchunk // NC
    cpw = cpc // NS
    rpw = n_pad // NS
    zrows = 128
    KB = 8

    def body(dst_hbm, degp_hbm, idx2d, ones_v, zb, dsem, hist_sp):
        c = lax.axis_index("c")
        s = lax.axis_index("s")
        pltpu.sync_copy(dst_hbm.at[pl.ds(c * cpc + s * cpw, cpw)], idx2d)

        _fill_const(ones_v, CH, jnp.ones((LANES,), jnp.float32))
        _fill_const(zb, zrows, jnp.zeros((LANES,), jnp.float32))

        def zinit(t, carry):
            pltpu.sync_copy(zb, hist_sp.at[pl.ds(s * rpw + t * zrows, zrows)])
            return carry

        lax.fori_loop(0, rpw // zrows, zinit, 0)
        plsc.subcore_barrier()

        def astep(g, carry):
            base = g * KB
            for b in range(KB):
                pltpu.async_copy(ones_v, hist_sp.at[idx2d.at[base + b]],
                                 dsem, add=True)
            for b in range(KB):
                pltpu.make_async_copy(ones_v, hist_sp.at[idx2d.at[base + b]],
                                      dsem).wait()
            return carry

        lax.fori_loop(0, cpw // KB, astep, 0)
        plsc.subcore_barrier()
        pltpu.sync_copy(hist_sp.at[pl.ds(s * rpw, rpw)],
                        degp_hbm.at[c].at[pl.ds(s * rpw, rpw)])

    return pl.kernel(
        body,
        out_type=jax.ShapeDtypeStruct((NC, n_pad, HW), jnp.float32),
        mesh=_sc_mesh(),
        scratch_types=[
            pltpu.VMEM((cpw, CH), jnp.int32),
            pltpu.VMEM((CH, HW), jnp.float32),
            pltpu.VMEM((zrows, HW), jnp.float32),
            pltpu.SemaphoreType.DMA,
            pltpu.VMEM_SHARED((n_pad, HW), jnp.float32),
        ],
    )(dst_ch)


def _dense_kernel(x, degp, Wc1, c1, W2, b2, Wg, n, n_pad, blk):
    d = x.shape[1]
    nblk = n_pad // blk

    def body(x_ref, degp_ref, wc1_ref, c1_ref, w2_ref, b2_ref, wg_ref,
             o1a, o1b, o2a, o2b):
        i = pl.program_id(0)
        deg = degp_ref[0, :, 0] + degp_ref[1, :, 0] + 1.0
        row = i * blk + lax.broadcasted_iota(jnp.int32, (blk, 1), 0)
        dis = jnp.where(row < n, lax.rsqrt(deg)[:, None], 0.0)
        xb = x_ref[...]
        t1 = (jnp.dot(xb, wc1_ref[...], preferred_element_type=jnp.float32)
              + c1_ref[...]) * dis
        h2 = jnp.dot(xb, w2_ref[...], preferred_element_type=jnp.float32) \
            + b2_ref[...]
        ss = jnp.sum(h2 * h2, axis=1, keepdims=True)
        h2n = h2 * (SCALE / jnp.maximum(jnp.sqrt(ss), 1e-12))
        t2 = jnp.dot(h2n, wg_ref[...], preferred_element_type=jnp.float32) \
            * dis
        h = d // 2
        o1a[...] = t1[:, :h]
        o1b[...] = t1[:, h:]
        o2a[...] = t2[:, :h]
        o2b[...] = t2[:, h:]

    out_sl = jax.ShapeDtypeStruct((n_pad, d // 2), jnp.float32)
    return pl.pallas_call(
        body,
        grid=(nblk,),
        in_specs=[
            pl.BlockSpec((blk, d), lambda i: (i, 0)),
            pl.BlockSpec((NC, blk, HW), lambda i: (0, i, 0)),
            pl.BlockSpec((d, d), lambda i: (0, 0)),
            pl.BlockSpec((1, d), lambda i: (0, 0)),
            pl.BlockSpec((d, d), lambda i: (0, 0)),
            pl.BlockSpec((1, d), lambda i: (0, 0)),
            pl.BlockSpec((d, d), lambda i: (0, 0)),
        ],
        out_specs=[pl.BlockSpec((blk, d // 2), lambda i: (i, 0))] * 4,
        out_shape=[out_sl] * 4,
    )(x, degp, Wc1, c1, W2, b2, Wg)


def _fold_kernel(W1, Wg, b1):
    d = W1.shape[0]

    def body(w1_ref, wg_ref, b1_ref, wc_ref, c1_ref):
        wg = wg_ref[...]
        wc_ref[...] = jnp.dot(w1_ref[...], wg, preferred_element_type=jnp.float32)
        c1_ref[...] = jnp.dot(b1_ref[...], wg, preferred_element_type=jnp.float32)

    return pl.pallas_call(
        body,
        out_shape=[
            jax.ShapeDtypeStruct((d, d), jnp.float32),
            jax.ShapeDtypeStruct((1, d), jnp.float32),
        ],
    )(W1, Wg, b1.reshape(1, d))


def _msg_kernel(src_ch, dst_ch, t0, t1, t2, t3, n_pad):
    nchunk = src_ch.shape[0]
    R = src_ch.shape[1]
    cpw = nchunk // NS
    rpw = n_pad // NS
    hw = t0.shape[1]
    NB = 4
    PH = 4
    pc = cpw // PH
    G = pc // NB

    def body(src_hbm, dst_hbm, t0_h, t1_h, t2_h, t3_h,
             a0_h, a1_h, a2_h, a3_h,
             sidx, didx, r0, r1, r2, r3, g0, g1, g2, g3,
             s0, s1, s2, s3, acc_sp):
        c = lax.axis_index("c")
        s = lax.axis_index("s")
        rows = (r0, r1, r2, r3)
        gsem = (g0, g1, g2, g3)
        ssem = (s0, s1, s2, s3)

        t_refs = (t0_h, t1_h, t2_h, t3_h)
        a_refs = (a0_h, a1_h, a2_h, a3_h)

        def run_slice(t_ref, a_ref):
            pltpu.sync_copy(t_ref.at[pl.ds(s * rpw, rpw)],
                            acc_sp.at[pl.ds(s * rpw, rpw)])
            plsc.subcore_barrier()

            for p in range(PH):
                base_hbm = s * cpw + p * pc
                pltpu.sync_copy(src_hbm.at[pl.ds(base_hbm, pc)], sidx)
                pltpu.sync_copy(dst_hbm.at[pl.ds(base_hbm, pc)], didx)

                for b in range(NB):
                    pltpu.async_copy(t_ref.at[sidx.at[b]], rows[b], gsem[b])

                def grp(g, carry):
                    base = g * NB
                    for b in range(NB):
                        j = base + b
                        pltpu.make_async_copy(t_ref.at[sidx.at[j]],
                                              rows[b], gsem[b]).wait()
                        pltpu.async_copy(rows[b], acc_sp.at[didx.at[j]],
                                         ssem[b], add=True)
                    for b in range(NB):
                        j = base + b
                        pltpu.make_async_copy(rows[b],
                                              acc_sp.at[didx.at[j]],
                                              ssem[b]).wait()

                        @pl.when(g < G - 1)
                        def _():
                            pltpu.async_copy(t_ref.at[sidx.at[j + NB]],
                                             rows[b], gsem[b])
                    return carry

                lax.fori_loop(0, G, grp, 0)

            plsc.subcore_barrier()
            pltpu.sync_copy(acc_sp.at[pl.ds(s * rpw, rpw)],
                            a_ref.at[pl.ds(s * rpw, rpw)])
            plsc.subcore_barrier()

        for sl in range(2):
            for cc in range(NC):
                @pl.when(c == cc)
                def _():
                    run_slice(t_refs[2 * cc + sl], a_refs[2 * cc + sl])

    out_sl = jax.ShapeDtypeStruct((n_pad, hw), jnp.float32)
    return pl.kernel(
        body,
        out_type=[out_sl] * 4,
        mesh=_sc_mesh(),
        scratch_types=[
            pltpu.VMEM((pc, R), jnp.int32),
            pltpu.VMEM((pc, R), jnp.int32),
            pltpu.VMEM((R, hw), jnp.float32),
            pltpu.VMEM((R, hw), jnp.float32),
            pltpu.VMEM((R, hw), jnp.float32),
            pltpu.VMEM((R, hw), jnp.float32),
            pltpu.SemaphoreType.DMA,
            pltpu.SemaphoreType.DMA,
            pltpu.SemaphoreType.DMA,
            pltpu.SemaphoreType.DMA,
            pltpu.SemaphoreType.DMA,
            pltpu.SemaphoreType.DMA,
            pltpu.SemaphoreType.DMA,
            pltpu.SemaphoreType.DMA,
            pltpu.VMEM_SHARED((n_pad, hw), jnp.float32),
        ],
    )(src_ch, dst_ch, t0, t1, t2, t3)


def _epilogue_kernel(a0, a1, a2, a3, degp, bg, n, n_pad, blk):
    hw = a0.shape[1]
    d = 2 * hw
    nblk = n_pad // blk

    def body(a0_r, a1_r, a2_r, a3_r, degp_ref, bg_ref, x_out, h_out):
        deg = degp_ref[0, :, 0] + degp_ref[1, :, 0] + 1.0
        dis = lax.rsqrt(deg)[:, None]
        bg = bg_ref[...]
        x_out[...] = dis * jnp.concatenate([a0_r[...], a1_r[...]], axis=1) + bg
        h_out[...] = dis * jnp.concatenate([a2_r[...], a3_r[...]], axis=1) + bg

    out = jax.ShapeDtypeStruct((n, d), jnp.float32)
    return pl.pallas_call(
        body,
        grid=(nblk,),
        in_specs=[pl.BlockSpec((blk, hw), lambda i: (i, 0))] * 4 + [
            pl.BlockSpec((NC, blk, HW), lambda i: (0, i, 0)),
            pl.BlockSpec((1, d), lambda i: (0, 0)),
        ],
        out_specs=[pl.BlockSpec((blk, d), lambda i: (i, 0))] * 2,
        out_shape=[out, out],
    )(a0, a1, a2, a3, degp, bg.reshape(1, d))


def kernel(x, edge_index, W1, b1, W2, b2, Wg, bg):
    n, d = x.shape
    e = edge_index.shape[1]
    blk = 256
    n_pad = ((n + blk - 1) // blk) * blk
    epw_align = NW * CH
    e_pad = ((e + epw_align - 1) // epw_align) * epw_align

    src = edge_index[0]
    dst = edge_index[1]
    padv = jnp.full((e_pad - e,), n, dtype=jnp.int32)
    src_p = jnp.concatenate([src, padv])
    dst_p = jnp.concatenate([dst, padv])

    dst_ch = dst_p.reshape(e_pad // CH, CH)
    degp = _deg_kernel(dst_ch, n_pad)

    Wc1, c1 = _fold_kernel(W1, Wg, b1)
    t0, t1, t2, t3 = _dense_kernel(x, degp, Wc1, c1, W2, b2.reshape(1, d),
                                   Wg, n, n_pad, blk)

    R3 = 64
    src_ch3 = src_p.reshape(e_pad // R3, R3)
    dst_ch3 = dst_p.reshape(e_pad // R3, R3)
    a0, a1, a2, a3 = _msg_kernel(src_ch3, dst_ch3, t0, t1, t2, t3, n_pad)

    x_, h = _epilogue_kernel(a0, a1, a2, a3, degp, bg, n, n_pad, blk)
    return (h, x_)
